# Initial kernel scaffold; baseline (speedup 1.0000x reference)
import functools

import jax
import jax.numpy as jnp
from jax import lax
from jax.experimental import pallas as pl
from jax.experimental.pallas import tpu as pltpu
from jax.experimental.pallas import tpu_sc as plsc

EMB_SIZE = 32

_info = plsc.get_sparse_core_info()
_NC, _NS = _info.num_cores, _info.num_subcores
_NW = _NC * _NS


def _make_gather(B: int, D: int, chunk: int):
    assert B % (_NW * chunk) == 0
    b_per_w = B // _NW
    n_chunks = b_per_w // chunk
    mesh = plsc.VectorSubcoreMesh(core_axis_name="c", subcore_axis_name="s")

    @functools.partial(
        pl.kernel,
        out_type=jax.ShapeDtypeStruct((B, D), jnp.float32),
        mesh=mesh,
        scratch_types=[
            pltpu.VMEM((chunk,), jnp.int32),
            pltpu.VMEM((chunk, D), jnp.float32),
            pltpu.SemaphoreType.DMA,
        ],
    )
    def gather_kernel(idx_hbm, table_hbm, out_hbm, idx_v, rows_v, sem):
        wid = lax.axis_index("s") * _NC + lax.axis_index("c")
        base = wid * b_per_w

        def body(i, carry):
            off = base + i * chunk
            pltpu.sync_copy(idx_hbm.at[pl.ds(off, chunk)], idx_v)
            pltpu.async_copy(table_hbm.at[idx_v], rows_v, sem).wait()
            pltpu.sync_copy(rows_v, out_hbm.at[pl.ds(off, chunk)])
            return carry

        lax.fori_loop(0, n_chunks, body, 0)

    return gather_kernel


def kernel(x, atom_emb_weight):
    orig_shape = x.shape
    B = x.size
    idx = x.reshape(-1).astype(jnp.int32)
    out = _make_gather(B, EMB_SIZE, 1280)(idx, atom_emb_weight)
    return out.reshape(*orig_shape, EMB_SIZE)


# speedup vs baseline: 2.9638x; 2.9638x over previous
import functools

import jax
import jax.numpy as jnp
from jax import lax
from jax.experimental import pallas as pl
from jax.experimental.pallas import tpu as pltpu
from jax.experimental.pallas import tpu_sc as plsc

EMB_SIZE = 32

_info = plsc.get_sparse_core_info()
_NC, _NS = _info.num_cores, _info.num_subcores
_NW = _NC * _NS


def _make_gather(B: int, D: int, chunk: int):
    assert B % (_NW * chunk) == 0
    b_per_w = B // _NW
    n_chunks = b_per_w // chunk
    mesh = plsc.VectorSubcoreMesh(core_axis_name="c", subcore_axis_name="s")

    @functools.partial(
        pl.kernel,
        out_type=jax.ShapeDtypeStruct((B, D), jnp.float32),
        mesh=mesh,
        scratch_types=[
            pltpu.VMEM((chunk,), jnp.int32),
            pltpu.VMEM((chunk, D), jnp.float32),
            pltpu.SemaphoreType.DMA,
        ],
        compiler_params=pltpu.CompilerParams(use_tc_tiling_on_sc=False),
    )
    def gather_kernel(idx_hbm, table_hbm, out_hbm, idx_v, rows_v, sem):
        wid = lax.axis_index("s") * _NC + lax.axis_index("c")
        base = wid * b_per_w

        def body(i, carry):
            off = base + i * chunk
            pltpu.sync_copy(idx_hbm.at[pl.ds(off, chunk)], idx_v)
            pltpu.async_copy(table_hbm.at[idx_v], rows_v, sem).wait()
            pltpu.sync_copy(rows_v, out_hbm.at[pl.ds(off, chunk)])
            return carry

        lax.fori_loop(0, n_chunks, body, 0)

    return gather_kernel


def kernel(x, atom_emb_weight):
    orig_shape = x.shape
    B = x.size
    idx = x.reshape(-1).astype(jnp.int32)
    out = _make_gather(B, EMB_SIZE, 1280)(idx, atom_emb_weight)
    return out.reshape(*orig_shape, EMB_SIZE)


# speedup vs baseline: 6.8859x; 2.3233x over previous
import functools

import jax
import jax.numpy as jnp
from jax import lax
from jax.experimental import pallas as pl
from jax.experimental.pallas import tpu as pltpu
from jax.experimental.pallas import tpu_sc as plsc

EMB_SIZE = 32

_info = plsc.get_sparse_core_info()
_NC, _NS = _info.num_cores, _info.num_subcores
_NW = _NC * _NS


def _make_gather(B: int, D: int, chunk: int):
    assert B % (_NW * chunk) == 0
    b_per_w = B // _NW
    n_chunks = b_per_w // chunk
    mesh = plsc.VectorSubcoreMesh(core_axis_name="c", subcore_axis_name="s")

    @functools.partial(
        pl.kernel,
        out_type=jax.ShapeDtypeStruct((B, D), jnp.float32),
        mesh=mesh,
        scratch_types=[
            pltpu.VMEM((chunk,), jnp.int32),
            pltpu.VMEM((chunk, D), jnp.float32),
            pltpu.SemaphoreType.DMA,
        ],
        compiler_params=pltpu.CompilerParams(use_tc_tiling_on_sc=False),
    )
    def gather_kernel(idx_hbm, table_hbm, out_hbm, idx_v, rows_v, sem):
        wid = lax.axis_index("s") * _NC + lax.axis_index("c")
        base = wid * b_per_w

        def body(i, carry):
            off = base + i * chunk
            pltpu.sync_copy(idx_hbm.at[pl.ds(off, chunk)], idx_v)
            pltpu.async_copy(table_hbm.at[idx_v], rows_v, sem).wait()
            pltpu.sync_copy(rows_v, out_hbm.at[pl.ds(off, chunk)])
            return carry

        lax.fori_loop(0, n_chunks, body, 0)

    return gather_kernel


def kernel(x, atom_emb_weight):
    n, m = x.shape
    B = x.size
    idx = x.T.reshape(-1).astype(jnp.int32)
    out = _make_gather(B, EMB_SIZE, 1280)(idx, atom_emb_weight)
    return out.reshape(m, n, EMB_SIZE).transpose(1, 0, 2)


# speedup vs baseline: 7.1778x; 1.0424x over previous
import functools

import jax
import jax.numpy as jnp
from jax import lax
from jax.experimental import pallas as pl
from jax.experimental.pallas import tpu as pltpu
from jax.experimental.pallas import tpu_sc as plsc

EMB_SIZE = 32

_info = plsc.get_sparse_core_info()
_NC, _NS = _info.num_cores, _info.num_subcores
_NW = _NC * _NS


def _make_gather(B: int, D: int, chunk: int, nbuf: int = 4, dist: int = 2):
    assert B % (_NW * chunk) == 0
    b_per_w = B // _NW
    n_chunks = b_per_w // chunk
    assert n_chunks % nbuf == 0 and dist < nbuf
    mesh = plsc.VectorSubcoreMesh(core_axis_name="c", subcore_axis_name="s")

    @functools.partial(
        pl.kernel,
        out_type=jax.ShapeDtypeStruct((B, D), jnp.float32),
        mesh=mesh,
        scratch_types=[
            pltpu.VMEM((b_per_w,), jnp.int32),
            *[pltpu.VMEM((chunk, D), jnp.float32) for _ in range(nbuf)],
            *[pltpu.SemaphoreType.DMA for _ in range(2 * nbuf)],
        ],
        compiler_params=pltpu.CompilerParams(use_tc_tiling_on_sc=False),
    )
    def gather_kernel(idx_hbm, table_hbm, out_hbm, idx_v, *scratch):
        rows = scratch[:nbuf]
        gsem = scratch[nbuf : 2 * nbuf]
        osem = scratch[2 * nbuf :]
        wid = lax.axis_index("s") * _NC + lax.axis_index("c")
        base = wid * b_per_w
        pltpu.sync_copy(idx_hbm.at[pl.ds(base, b_per_w)], idx_v)

        def g_start(i, b):
            pltpu.async_copy(
                table_hbm.at[idx_v.at[pl.ds(i * chunk, chunk)]], rows[b], gsem[b]
            )

        def g_wait(i, b):
            pltpu.make_async_copy(
                table_hbm.at[idx_v.at[pl.ds(i * chunk, chunk)]], rows[b], gsem[b]
            ).wait()

        def o_start(i, b):
            pltpu.async_copy(
                rows[b], out_hbm.at[pl.ds(base + i * chunk, chunk)], osem[b]
            )

        def o_wait(i, b):
            pltpu.make_async_copy(
                rows[b], out_hbm.at[pl.ds(base + i * chunk, chunk)], osem[b]
            ).wait()

        for b in range(dist):
            g_start(b, b)

        def body(g, carry):
            for b in range(nbuf):
                i = g * nbuf + b
                g_wait(i, b)
                o_start(i, b)
                nxt = i + dist
                c = (b + dist) % nbuf

                @pl.when(nxt < n_chunks)
                def _():
                    @pl.when(nxt >= nbuf)
                    def _():
                        o_wait(nxt - nbuf, c)

                    g_start(nxt, c)

            return carry

        lax.fori_loop(0, n_chunks // nbuf, body, 0)
        for b in range(nbuf):
            o_wait(n_chunks - nbuf + b, b)

    return gather_kernel


def kernel(x, atom_emb_weight):
    n, m = x.shape
    B = x.size
    idx = x.T.reshape(-1).astype(jnp.int32)
    out = _make_gather(B, EMB_SIZE, 640)(idx, atom_emb_weight)
    return out.reshape(m, n, EMB_SIZE).transpose(1, 0, 2)
